# baseline (speedup 1.0000x reference)
import functools
import jax
import jax.numpy as jnp
from jax import lax
from jax.experimental import pallas as pl
from jax.experimental.pallas import tpu as pltpu
from jax.experimental.pallas import tpu_sc as plsc

B = 32
L = 100
D_PAD = 10240
F_PAD = 640
FB = 32
NFB = F_PAD // FB
NW = 32
DT = D_PAD // NW
NJ = DT // 16


def _sc_body(xT_hbm, id_hbm, lvl_hbm, out_hbm, xb_v, idx_v, lvl_v, id_v, acc_v):
    cid = lax.axis_index("c")
    sid = lax.axis_index("s")
    wid = sid * 2 + cid

    pltpu.sync_copy(lvl_hbm.at[wid], lvl_v)

    zero = jnp.zeros((16,), jnp.float32)

    def zero_body(i, _):
        b = i // NJ
        j = i % NJ
        acc_v[b, pl.ds(16 * j, 16)] = zero
        return 0

    lax.fori_loop(0, B * NJ, zero_body, 0)

    lane_iota = lax.broadcasted_iota(jnp.int32, (16,), 0)

    def fb_body(fb, _):
        f0 = fb * FB
        pltpu.sync_copy(xT_hbm.at[pl.ds(f0 * B, FB * B)], xb_v)
        pltpu.sync_copy(id_hbm.at[wid, pl.ds(f0, FB), :], id_v)

        def idx_body(i, _):
            v = xb_v[pl.ds(i * 16, 16)]
            idx_v[pl.ds(i * 16, 16)] = jnp.clip(
                (v * float(L)).astype(jnp.int32), 0, L - 1)
            return 0

        lax.fori_loop(0, FB * 2, idx_body, 0)

        def fl_body(fl, _):
            idrow = [id_v[fl, pl.ds(16 * j, 16)] for j in range(NJ)]

            def b_body(b, _):
                pos = jnp.full((16,), fl * B + b, jnp.int32)
                row = plsc.load_gather(idx_v, [pos])
                for j in range(NJ):
                    lv = plsc.load_gather(lvl_v, [row, lane_iota + 16 * j])
                    plsc.addupdate(acc_v.at[b, pl.ds(16 * j, 16)], lv * idrow[j])
                return 0

            lax.fori_loop(0, B, b_body, 0)
            return 0

        lax.fori_loop(0, FB, fl_body, 0)
        return 0

    lax.fori_loop(0, NFB, fb_body, 0)

    one = jnp.full((16,), 1.0, jnp.float32)
    neg = jnp.full((16,), -1.0, jnp.float32)

    def q_body(i, _):
        b = i // NJ
        j = i % NJ
        s = acc_v[b, pl.ds(16 * j, 16)]
        acc_v[b, pl.ds(16 * j, 16)] = jnp.where(s > 0, one, neg)
        return 0

    lax.fori_loop(0, B * NJ, q_body, 0)
    pltpu.sync_copy(acc_v, out_hbm.at[wid])


def _classify_body(q_ref, cw_ref, out_ref):
    out_ref[...] = jax.lax.dot_general(
        q_ref[...], cw_ref[...],
        (((1,), (1,)), ((), ())),
        preferred_element_type=jnp.float32,
    )


def kernel(x, id_weight, level_weight, classify_weight):
    F = x.shape[1]
    D = level_weight.shape[1]
    C = classify_weight.shape[0]
    xT = jnp.pad(x.T, ((0, F_PAD - F), (0, 0))).reshape(-1)
    id_p = jnp.pad(id_weight, ((0, F_PAD - F), (0, D_PAD - D)))
    id_r = id_p.reshape(F_PAD, NW, DT).transpose(1, 0, 2)
    lvl_p = jnp.pad(level_weight, ((0, 0), (0, D_PAD - D)))
    lvl_r = lvl_p.reshape(L, NW, DT).transpose(1, 0, 2)
    cw_p = jnp.pad(classify_weight, ((0, 0), (0, D_PAD - D)))

    mesh = plsc.VectorSubcoreMesh(core_axis_name="c", subcore_axis_name="s")
    sc = functools.partial(
        pl.kernel,
        out_type=jax.ShapeDtypeStruct((NW, B, DT), jnp.float32),
        mesh=mesh,
        compiler_params=pltpu.CompilerParams(needs_layout_passes=False),
        scratch_types=[
            pltpu.VMEM((FB * B,), jnp.float32),
            pltpu.VMEM((FB * B,), jnp.int32),
            pltpu.VMEM((L, DT), jnp.float32),
            pltpu.VMEM((FB, DT), jnp.float32),
            pltpu.VMEM((B, DT), jnp.float32),
        ],
    )(_sc_body)
    q = sc(xT, id_r, lvl_r).transpose(1, 0, 2).reshape(B, D_PAD)

    logit = pl.pallas_call(
        _classify_body,
        in_specs=[
            pl.BlockSpec((B, D_PAD), lambda: (0, 0)),
            pl.BlockSpec((C, D_PAD), lambda: (0, 0)),
        ],
        out_specs=pl.BlockSpec((B, C), lambda: (0, 0)),
        out_shape=jax.ShapeDtypeStruct((B, C), jnp.float32),
    )(q, cw_p)
    return logit


# speedup vs baseline: 2.6711x; 2.6711x over previous
import functools
import jax
import jax.numpy as jnp
from jax import lax
from jax.experimental import pallas as pl
from jax.experimental.pallas import tpu as pltpu
from jax.experimental.pallas import tpu_sc as plsc

B = 32
L = 100
D_PAD = 10240
F_PAD = 640
FB = 32
NFB = F_PAD // FB
NW = 32
DT = D_PAD // NW
NJ = DT // 16


def _sc_body(xT_hbm, id_hbm, lvl_hbm, out_hbm, xb_v, idx_v, lvl_v, id_v, acc_v):
    cid = lax.axis_index("c")
    sid = lax.axis_index("s")
    wid = sid * 2 + cid

    pltpu.sync_copy(lvl_hbm.at[wid], lvl_v)

    zero = jnp.zeros((16,), jnp.float32)

    def zero_body(i, _):
        b = i // NJ
        j = i % NJ
        acc_v[b, pl.ds(16 * j, 16)] = zero
        return 0

    lax.fori_loop(0, B * NJ, zero_body, 0)

    lane_iota = lax.broadcasted_iota(jnp.int32, (16,), 0)

    def fb_body(fb, _):
        f0 = fb * FB
        pltpu.sync_copy(xT_hbm.at[pl.ds(f0 * B, FB * B)], xb_v)
        pltpu.sync_copy(id_hbm.at[wid, pl.ds(f0, FB), :], id_v)

        def idx_body(i, _):
            v = xb_v[pl.ds(i * 16, 16)]
            idx_v[pl.ds(i * 16, 16)] = jnp.clip(
                (v * float(L)).astype(jnp.int32), 0, L - 1)
            return 0

        lax.fori_loop(0, FB * 2, idx_body, 0)

        def fl_body(fl, _):
            idrow = [id_v[fl, pl.ds(16 * j, 16)] for j in range(NJ)]

            @plsc.parallel_loop(0, B, unroll=4)
            def b_body(b):
                pos = jnp.full((16,), fl * B + b, jnp.int32)
                row = plsc.load_gather(idx_v, [pos])
                for j in range(NJ):
                    lv = plsc.load_gather(lvl_v, [row, lane_iota + 16 * j])
                    plsc.addupdate(acc_v.at[b, pl.ds(16 * j, 16)], lv * idrow[j])

            return 0

        lax.fori_loop(0, FB, fl_body, 0)
        return 0

    lax.fori_loop(0, NFB, fb_body, 0)

    one = jnp.full((16,), 1.0, jnp.float32)
    neg = jnp.full((16,), -1.0, jnp.float32)

    def q_body(i, _):
        b = i // NJ
        j = i % NJ
        s = acc_v[b, pl.ds(16 * j, 16)]
        acc_v[b, pl.ds(16 * j, 16)] = jnp.where(s > 0, one, neg)
        return 0

    lax.fori_loop(0, B * NJ, q_body, 0)
    pltpu.sync_copy(acc_v, out_hbm.at[wid])


def _classify_body(q_ref, cw_ref, out_ref):
    out_ref[...] = jax.lax.dot_general(
        q_ref[...], cw_ref[...],
        (((1,), (1,)), ((), ())),
        preferred_element_type=jnp.float32,
    )


def kernel(x, id_weight, level_weight, classify_weight):
    F = x.shape[1]
    D = level_weight.shape[1]
    C = classify_weight.shape[0]
    xT = jnp.pad(x.T, ((0, F_PAD - F), (0, 0))).reshape(-1)
    id_p = jnp.pad(id_weight, ((0, F_PAD - F), (0, D_PAD - D)))
    id_r = id_p.reshape(F_PAD, NW, DT).transpose(1, 0, 2)
    lvl_p = jnp.pad(level_weight, ((0, 0), (0, D_PAD - D)))
    lvl_r = lvl_p.reshape(L, NW, DT).transpose(1, 0, 2)
    cw_p = jnp.pad(classify_weight, ((0, 0), (0, D_PAD - D)))

    mesh = plsc.VectorSubcoreMesh(core_axis_name="c", subcore_axis_name="s")
    sc = functools.partial(
        pl.kernel,
        out_type=jax.ShapeDtypeStruct((NW, B, DT), jnp.float32),
        mesh=mesh,
        compiler_params=pltpu.CompilerParams(needs_layout_passes=False),
        scratch_types=[
            pltpu.VMEM((FB * B,), jnp.float32),
            pltpu.VMEM((FB * B,), jnp.int32),
            pltpu.VMEM((L, DT), jnp.float32),
            pltpu.VMEM((FB, DT), jnp.float32),
            pltpu.VMEM((B, DT), jnp.float32),
        ],
    )(_sc_body)
    q = sc(xT, id_r, lvl_r).transpose(1, 0, 2).reshape(B, D_PAD)

    logit = pl.pallas_call(
        _classify_body,
        in_specs=[
            pl.BlockSpec((B, D_PAD), lambda: (0, 0)),
            pl.BlockSpec((C, D_PAD), lambda: (0, 0)),
        ],
        out_specs=pl.BlockSpec((B, C), lambda: (0, 0)),
        out_shape=jax.ShapeDtypeStruct((B, C), jnp.float32),
    )(q, cw_p)
    return logit
